# baseline (speedup 1.0000x reference)
import functools

import jax
import jax.numpy as jnp
import numpy as np
from jax import lax
from jax.experimental import pallas as pl
from jax.experimental.pallas import tpu as pltpu
from jax.experimental.pallas import tpu_sc as plsc

_N, _E, _D, _OUT, _PROP = 10000, 320000, 128, 7, 5
_EPS = float(np.finfo(np.float32).eps)

_NC, _NS = 2, 16
_NW = _NC * _NS
_B = 128
_NBB = 80
_EW = _NBB * _B
_EPAD = _EW * _NW
_NPAD = 10240
_RPT = _NPAD // _NS


def _zero_fill_vmem(ref, rows, cols):
    zero16 = jnp.zeros((16,), jnp.float32)
    for r in range(rows):
        for c in range(cols // 16):
            ref[r, pl.ds(c * 16, 16)] = zero16


def _agg_step_body(state_h, src_h, dst_h, pp_h,
                   acc_sh, sidx_v, b0, b1, i0, i1, zrow_v,
                   g0, g1, s0, s1, zsem):
    cid = lax.axis_index("c")
    sid = lax.axis_index("s")
    wid = sid * jnp.int32(_NC) + cid
    base = sid * jnp.int32(_RPT)

    bufs = (b0, b1)
    ibufs = (i0, i1)
    gsems = (g0, g1)
    ssems = (s0, s1)

    pltpu.sync_copy(src_h.at[pl.ds(wid * jnp.int32(_EW), _EW)], sidx_v)

    _zero_fill_vmem(zrow_v, 32, _D)

    def zbody(k, _):
        pltpu.async_copy(zrow_v, acc_sh.at[pl.ds(base + k * jnp.int32(32), 32)],
                         zsem)
        return jnp.int32(0)
    lax.fori_loop(jnp.int32(0), jnp.int32(_RPT // 32), zbody, jnp.int32(0))

    def zdrain(k, _):
        pltpu.make_async_copy(
            zrow_v, acc_sh.at[pl.ds(base, 32)], zsem).wait()
        return jnp.int32(0)
    lax.fori_loop(jnp.int32(0), jnp.int32(_RPT // 32), zdrain, jnp.int32(0))
    plsc.subcore_barrier()

    def start_gather(j, u):
        pltpu.async_copy(
            dst_h.at[pl.ds(wid * jnp.int32(_EW) + j * jnp.int32(_B), _B)],
            ibufs[u], gsems[u])
        pltpu.async_copy(
            state_h.at[sidx_v.at[pl.ds(j * jnp.int32(_B), _B)]],
            bufs[u], gsems[u])

    def wait_gather(u):
        pltpu.make_async_copy(
            dst_h.at[pl.ds(jnp.int32(0), _B)], ibufs[u], gsems[u]).wait()
        pltpu.make_async_copy(
            state_h.at[sidx_v.at[pl.ds(jnp.int32(0), _B)]],
            bufs[u], gsems[u]).wait()

    def start_scatter(j, u):
        pltpu.async_copy(bufs[u], acc_sh.at[ibufs[u]], ssems[u], add=True)

    def wait_scatter(u):
        pltpu.make_async_copy(bufs[u], acc_sh.at[ibufs[u]], ssems[u]).wait()

    def body(k, _):
        for u in range(2):
            j = k * jnp.int32(2) + jnp.int32(u)

            @pl.when(k >= jnp.int32(1))
            def _():
                wait_scatter(u)
            start_gather(j, u)
            jb = j - jnp.int32(1)
            ub = 1 - u

            @pl.when(jb >= jnp.int32(0))
            def _():
                wait_gather(ub)
                start_scatter(jb, ub)
        return jnp.int32(0)
    lax.fori_loop(jnp.int32(0), jnp.int32(_NBB // 2), body, jnp.int32(0))

    wait_gather(1)
    start_scatter(jnp.int32(_NBB - 1), 1)
    for u in range(2):
        wait_scatter(u)
    plsc.subcore_barrier()

    def obody(k, _):
        roff = base + k * jnp.int32(32)
        pltpu.sync_copy(acc_sh.at[pl.ds(roff, 32)], zrow_v)
        pltpu.sync_copy(zrow_v, pp_h.at[cid, pl.ds(roff, 32)])
        return jnp.int32(0)
    lax.fori_loop(jnp.int32(0), jnp.int32(_RPT // 32), obody, jnp.int32(0))


def _degrees_body(dst_h, cc_h, acc_sh, didx_v, j0, j1, ones_v, zs_v,
                  s0, s1):
    cid = lax.axis_index("c")
    sid = lax.axis_index("s")
    wid = sid * jnp.int32(_NC) + cid
    base = sid * jnp.int32(_RPT)

    ibufs = (j0, j1)
    ssems = (s0, s1)

    zero16 = jnp.zeros((16,), jnp.float32)
    one16 = jnp.ones((16,), jnp.float32)
    for k in range(-(-_RPT // 16)):
        zs_v[pl.ds(min(k * 16, _RPT - 16), 16)] = zero16
    for k in range(_B // 16):
        ones_v[pl.ds(k * 16, 16)] = one16

    pltpu.sync_copy(zs_v, acc_sh.at[pl.ds(base, _RPT)])

    def ldbody(k, _):
        off = k * jnp.int32(1024)
        pltpu.sync_copy(dst_h.at[pl.ds(wid * jnp.int32(_EW) + off, 1024)],
                        didx_v.at[pl.ds(off, 1024)])
        return jnp.int32(0)
    lax.fori_loop(jnp.int32(0), jnp.int32(_EW // 1024), ldbody, jnp.int32(0))
    plsc.subcore_barrier()

    def start_scatter(j, u):
        for t in range(_B // 16):
            ibufs[u][pl.ds(t * 16, 16)] = didx_v[
                pl.ds(j * jnp.int32(_B) + jnp.int32(t * 16), 16)]
        pltpu.async_copy(ones_v, acc_sh.at[ibufs[u]], ssems[u], add=True)

    def wait_scatter(u):
        pltpu.make_async_copy(ones_v, acc_sh.at[ibufs[u]], ssems[u]).wait()

    def body(k, _):
        for u in range(2):
            j = k * jnp.int32(2) + jnp.int32(u)

            @pl.when(k >= jnp.int32(1))
            def _():
                wait_scatter(u)
            start_scatter(j, u)
        return jnp.int32(0)
    lax.fori_loop(jnp.int32(0), jnp.int32(_NBB // 2), body, jnp.int32(0))
    for u in range(2):
        wait_scatter(u)
    plsc.subcore_barrier()

    obase = cid * jnp.int32(_NPAD) + base
    pltpu.sync_copy(acc_sh.at[pl.ds(base, _RPT)], zs_v)
    pltpu.sync_copy(zs_v, cc_h.at[pl.ds(obase, _RPT)])


@functools.lru_cache(maxsize=1)
def _sc_kernels():
    mesh = plsc.VectorSubcoreMesh(core_axis_name="c", subcore_axis_name="s",
                                  num_cores=_NC, num_subcores=_NS)
    agg = pl.kernel(
        _agg_step_body,
        out_type=jax.ShapeDtypeStruct((2, _NPAD, _D), jnp.float32),
        mesh=mesh,
        scratch_types=(
            pltpu.VMEM_SHARED((_NPAD, _D), jnp.float32),
            pltpu.VMEM((_EW,), jnp.int32),
            pltpu.VMEM((_B, _D), jnp.float32),
            pltpu.VMEM((_B, _D), jnp.float32),
            pltpu.VMEM((_B,), jnp.int32),
            pltpu.VMEM((_B,), jnp.int32),
            pltpu.VMEM((32, _D), jnp.float32),
            pltpu.SemaphoreType.DMA,
            pltpu.SemaphoreType.DMA,
            pltpu.SemaphoreType.DMA,
            pltpu.SemaphoreType.DMA,
            pltpu.SemaphoreType.DMA,
        ),
    )
    deg = pl.kernel(
        _degrees_body,
        out_type=jax.ShapeDtypeStruct((2 * _NPAD,), jnp.float32),
        mesh=mesh,
        scratch_types=(
            pltpu.VMEM_SHARED((_NPAD,), jnp.float32),
            pltpu.VMEM((_EW,), jnp.int32),
            pltpu.VMEM((_B,), jnp.int32),
            pltpu.VMEM((_B,), jnp.int32),
            pltpu.VMEM((_B,), jnp.float32),
            pltpu.VMEM((_RPT,), jnp.float32),
            pltpu.SemaphoreType.DMA,
            pltpu.SemaphoreType.DMA,
        ),
    )
    return agg, deg


_BLK = 400
_BLKG = 1000


def _gru_body(p0, p1, c0, c1, h, wih, whh, bih, bhh, out_h, out_sq):
    i = pl.program_id(0)
    cnt = c0[...] + c1[...]
    agg = (p0[0] + p1[0]) / (cnt + jnp.float32(_EPS))
    hh = h[...]
    dn = (((1,), (1,)), ((), ()))
    gi = lax.dot_general(agg, wih[...], dn, preferred_element_type=jnp.float32)
    gi = gi + bih[...]
    gh = lax.dot_general(hh, whh[...], dn, preferred_element_type=jnp.float32)
    gh = gh + bhh[...]
    r = jax.nn.sigmoid(gi[:, :_D] + gh[:, :_D])
    z = jax.nn.sigmoid(gi[:, _D:2 * _D] + gh[:, _D:2 * _D])
    n = jnp.tanh(gi[:, 2 * _D:] + r * gh[:, 2 * _D:])
    new = (jnp.float32(1.0) - z) * n + z * hh
    out_h[...] = new

    @pl.when(i == 0)
    def _():
        out_sq[...] = jnp.zeros((1, 1), jnp.float32)
    out_sq[...] += jnp.sum((new - hh) ** 2).reshape(1, 1)


def _gru_step(p0, p1, c0, c1, h, wih, whh, bih, bhh):
    row = lambda i: (i, i * 0)
    p0m = lambda i: (i * 0, i, i * 0)
    p1m = lambda i: (i * 0 + 1, i, i * 0)
    fixed = lambda i: (i * 0, i * 0)
    return pl.pallas_call(
        _gru_body,
        grid=(_N // _BLKG,),
        in_specs=[
            pl.BlockSpec((1, _BLKG, _D), p0m),
            pl.BlockSpec((1, _BLKG, _D), p1m),
            pl.BlockSpec((_BLKG, 1), row),
            pl.BlockSpec((_BLKG, 1), row),
            pl.BlockSpec((_BLKG, _D), row),
            pl.BlockSpec((3 * _D, _D), fixed),
            pl.BlockSpec((3 * _D, _D), fixed),
            pl.BlockSpec((1, 3 * _D), fixed),
            pl.BlockSpec((1, 3 * _D), fixed),
        ],
        out_specs=[
            pl.BlockSpec((_BLKG, _D), row),
            pl.BlockSpec((1, 1), fixed),
        ],
        out_shape=[
            jax.ShapeDtypeStruct((_N, _D), jnp.float32),
            jax.ShapeDtypeStruct((1, 1), jnp.float32),
        ],
    )(p0, p1, c0, c1, h, wih, whh, bih, bhh)


def _head_body(h, wo, bo, tg, mk, y_h, ps, ms):
    i = pl.program_id(0)
    dn = (((1,), (1,)), ((), ()))
    y = lax.dot_general(h[...], wo[...], dn, preferred_element_type=jnp.float32)
    y = y + bo[...]
    mkf = mk[...]
    ym = y * mkf
    col = lax.broadcasted_iota(jnp.int32, (_BLK, _D), 1)
    valid = col < _OUT
    yv = jnp.where(valid, ym, jnp.float32(-1e30))
    m = jnp.max(yv, axis=1, keepdims=True)
    e = jnp.where(valid, jnp.exp(ym - m), jnp.float32(0.0))
    se = jnp.sum(e, axis=1, keepdims=True)
    logp = ym - m - jnp.log(se)
    picked = jnp.sum(jnp.where(col == tg[...], logp, jnp.float32(0.0)), axis=1, keepdims=True)
    y_h[...] = jnp.where(valid, ym, jnp.float32(0.0))

    @pl.when(i == 0)
    def _():
        ps[...] = jnp.zeros((1, 1), jnp.float32)
        ms[...] = jnp.zeros((1, 1), jnp.float32)
    ps[...] += jnp.sum(picked * mkf).reshape(1, 1)
    ms[...] += jnp.sum(mkf).reshape(1, 1)


def _head(h, wo, bo, tg, mk):
    row = lambda i: (i, i * 0)
    fixed = lambda i: (i * 0, i * 0)
    return pl.pallas_call(
        _head_body,
        grid=(_N // _BLK,),
        in_specs=[
            pl.BlockSpec((_BLK, _D), row),
            pl.BlockSpec((_D, _D), fixed),
            pl.BlockSpec((1, _D), fixed),
            pl.BlockSpec((_BLK, 1), row),
            pl.BlockSpec((_BLK, 1), row),
        ],
        out_specs=[
            pl.BlockSpec((_BLK, _D), row),
            pl.BlockSpec((1, 1), fixed),
            pl.BlockSpec((1, 1), fixed),
        ],
        out_shape=[
            jax.ShapeDtypeStruct((_N, _D), jnp.float32),
            jax.ShapeDtypeStruct((1, 1), jnp.float32),
            jax.ShapeDtypeStruct((1, 1), jnp.float32),
        ],
    )(h, wo, bo, tg, mk)


def _to_f64(x):
    b = jax.lax.bitcast_convert_type(x, jnp.uint32).astype(jnp.uint64)
    sign = (b >> 31) << 63
    expf = (b >> 23) & jnp.uint64(0xFF)
    man = b & jnp.uint64(0x7FFFFF)
    norm = expf != 0
    exp64 = jnp.where(norm, expf + jnp.uint64(896), jnp.uint64(0)) << 52
    man64 = jnp.where(norm, man, jnp.uint64(0)) << 29
    return jax.lax.bitcast_convert_type(sign | exp64 | man64, jnp.float64)


def kernel(edge, feat, target, mask, W_ih, W_hh, b_ih, b_hh, W_out, b_out):
    e = edge[0].astype(jnp.int32)
    npe = _EPAD - _E
    ar = jnp.arange(npe, dtype=jnp.int32)
    src = jnp.concatenate([e[:, 0], ar % _N])
    dst1 = jnp.concatenate([e[:, 1], _N + (ar % (_NPAD - _N))])
    state = feat[0].astype(jnp.float32)
    W_ih = W_ih.astype(jnp.float32)
    W_hh = W_hh.astype(jnp.float32)
    W_out = W_out.astype(jnp.float32)
    b_out = b_out.astype(jnp.float32)
    tgt64 = target[0]
    mk = mask[0]
    mkf = mk.astype(jnp.float32).reshape(_N, 1)
    tgm = jnp.where(mk, tgt64, 0).astype(jnp.int32).reshape(_N, 1)

    _agg_step, _degrees = _sc_kernels()
    cc = _degrees(dst1)
    c0 = cc[:_N].reshape(_N, 1)
    c1 = cc[_NPAD:_NPAD + _N].reshape(_N, 1)

    bih = b_ih.astype(jnp.float32).reshape(1, 3 * _D)
    bhh = b_hh.astype(jnp.float32).reshape(1, 3 * _D)
    wo = jnp.pad(W_out, ((0, _D - _OUT), (0, 0)))
    bo = jnp.pad(b_out, (0, _D - _OUT)).reshape(1, _D)

    diffs = []
    for _ in range(_PROP):
        pp = _agg_step(state, src, dst1)
        new_state, sq = _gru_step(pp, pp, c0, c1, state,
                                  W_ih, W_hh, bih, bhh)
        diffs.append(jnp.sqrt(sq[0, 0]) / float(_N * _D))
        state = new_state

    ym_full, ps, ms = _head(state, wo, bo, tgm, mkf)
    y_m = _to_f64(ym_full[:, :_OUT])
    t_m = jnp.where(mk, tgt64, jnp.zeros_like(tgt64))
    loss = -(ps[0, 0] / ms[0, 0]).astype(jnp.float64)
    return y_m, t_m, jnp.stack(diffs).astype(jnp.float64), loss


# speedup vs baseline: 1.0396x; 1.0396x over previous
import functools

import jax
import jax.numpy as jnp
import numpy as np
from jax import lax
from jax.experimental import pallas as pl
from jax.experimental.pallas import tpu as pltpu
from jax.experimental.pallas import tpu_sc as plsc

_N, _E, _D, _OUT, _PROP = 10000, 320000, 128, 7, 5
_EPS = float(np.finfo(np.float32).eps)

_NC, _NS = 2, 16
_NW = _NC * _NS
_B = 128
_NBB = 80
_EW = _NBB * _B
_EPAD = _EW * _NW
_NPAD = 10240
_RPT = _NPAD // _NS


def _zero_fill_vmem(ref, rows, cols):
    zero16 = jnp.zeros((16,), jnp.float32)
    for r in range(rows):
        for c in range(cols // 16):
            ref[r, pl.ds(c * 16, 16)] = zero16


def _agg_step_body(state_h, src_h, dst_h, pp_h,
                   acc_sh, sidx_v, b0, b1, i0, i1, zrow_v,
                   g0, g1, s0, s1, zsem):
    cid = lax.axis_index("c")
    sid = lax.axis_index("s")
    wid = sid * jnp.int32(_NC) + cid
    base = sid * jnp.int32(_RPT)

    bufs = (b0, b1)
    ibufs = (i0, i1)
    gsems = (g0, g1)
    ssems = (s0, s1)

    pltpu.sync_copy(src_h.at[pl.ds(wid * jnp.int32(_EW), _EW)], sidx_v)

    _zero_fill_vmem(zrow_v, 32, _D)

    def zbody(k, _):
        pltpu.async_copy(zrow_v, acc_sh.at[pl.ds(base + k * jnp.int32(32), 32)],
                         zsem)
        return jnp.int32(0)
    lax.fori_loop(jnp.int32(0), jnp.int32(_RPT // 32), zbody, jnp.int32(0))

    def zdrain(k, _):
        pltpu.make_async_copy(
            zrow_v, acc_sh.at[pl.ds(base, 32)], zsem).wait()
        return jnp.int32(0)
    lax.fori_loop(jnp.int32(0), jnp.int32(_RPT // 32), zdrain, jnp.int32(0))
    plsc.subcore_barrier()

    def start_gather(j, u):
        pltpu.async_copy(
            dst_h.at[pl.ds(wid * jnp.int32(_EW) + j * jnp.int32(_B), _B)],
            ibufs[u], gsems[u])
        pltpu.async_copy(
            state_h.at[sidx_v.at[pl.ds(j * jnp.int32(_B), _B)]],
            bufs[u], gsems[u])

    def wait_gather(u):
        pltpu.make_async_copy(
            dst_h.at[pl.ds(jnp.int32(0), _B)], ibufs[u], gsems[u]).wait()
        pltpu.make_async_copy(
            state_h.at[sidx_v.at[pl.ds(jnp.int32(0), _B)]],
            bufs[u], gsems[u]).wait()

    def start_scatter(j, u):
        pltpu.async_copy(bufs[u], acc_sh.at[ibufs[u]], ssems[u], add=True)

    def wait_scatter(u):
        pltpu.make_async_copy(bufs[u], acc_sh.at[ibufs[u]], ssems[u]).wait()

    def body(k, _):
        for u in range(2):
            j = k * jnp.int32(2) + jnp.int32(u)

            @pl.when(k >= jnp.int32(1))
            def _():
                wait_scatter(u)
            start_gather(j, u)
            jb = j - jnp.int32(1)
            ub = 1 - u

            @pl.when(jb >= jnp.int32(0))
            def _():
                wait_gather(ub)
                start_scatter(jb, ub)
        return jnp.int32(0)
    lax.fori_loop(jnp.int32(0), jnp.int32(_NBB // 2), body, jnp.int32(0))

    wait_gather(1)
    start_scatter(jnp.int32(_NBB - 1), 1)
    for u in range(2):
        wait_scatter(u)
    plsc.subcore_barrier()

    def obody(k, _):
        roff = base + k * jnp.int32(32)
        pltpu.sync_copy(acc_sh.at[pl.ds(roff, 32)], zrow_v)
        pltpu.sync_copy(zrow_v, pp_h.at[cid, pl.ds(roff, 32)])
        return jnp.int32(0)
    lax.fori_loop(jnp.int32(0), jnp.int32(_RPT // 32), obody, jnp.int32(0))


def _degrees_body(dst_h, cc_h, acc_sh, didx_v, j0, j1, ones_v, zs_v,
                  s0, s1):
    cid = lax.axis_index("c")
    sid = lax.axis_index("s")
    wid = sid * jnp.int32(_NC) + cid
    base = sid * jnp.int32(_RPT)

    ibufs = (j0, j1)
    ssems = (s0, s1)

    zero16 = jnp.zeros((16,), jnp.float32)
    one16 = jnp.ones((16,), jnp.float32)
    for k in range(-(-_RPT // 16)):
        zs_v[pl.ds(min(k * 16, _RPT - 16), 16)] = zero16
    for k in range(_B // 16):
        ones_v[pl.ds(k * 16, 16)] = one16

    pltpu.sync_copy(zs_v, acc_sh.at[pl.ds(base, _RPT)])

    def ldbody(k, _):
        off = k * jnp.int32(1024)
        pltpu.sync_copy(dst_h.at[pl.ds(wid * jnp.int32(_EW) + off, 1024)],
                        didx_v.at[pl.ds(off, 1024)])
        return jnp.int32(0)
    lax.fori_loop(jnp.int32(0), jnp.int32(_EW // 1024), ldbody, jnp.int32(0))
    plsc.subcore_barrier()

    def start_scatter(j, u):
        for t in range(_B // 16):
            ibufs[u][pl.ds(t * 16, 16)] = didx_v[
                pl.ds(j * jnp.int32(_B) + jnp.int32(t * 16), 16)]
        pltpu.async_copy(ones_v, acc_sh.at[ibufs[u]], ssems[u], add=True)

    def wait_scatter(u):
        pltpu.make_async_copy(ones_v, acc_sh.at[ibufs[u]], ssems[u]).wait()

    def body(k, _):
        for u in range(2):
            j = k * jnp.int32(2) + jnp.int32(u)

            @pl.when(k >= jnp.int32(1))
            def _():
                wait_scatter(u)
            start_scatter(j, u)
        return jnp.int32(0)
    lax.fori_loop(jnp.int32(0), jnp.int32(_NBB // 2), body, jnp.int32(0))
    for u in range(2):
        wait_scatter(u)
    plsc.subcore_barrier()

    obase = cid * jnp.int32(_NPAD) + base
    pltpu.sync_copy(acc_sh.at[pl.ds(base, _RPT)], zs_v)
    pltpu.sync_copy(zs_v, cc_h.at[pl.ds(obase, _RPT)])


@functools.lru_cache(maxsize=1)
def _sc_kernels():
    mesh = plsc.VectorSubcoreMesh(core_axis_name="c", subcore_axis_name="s",
                                  num_cores=_NC, num_subcores=_NS)
    agg = pl.kernel(
        _agg_step_body,
        out_type=jax.ShapeDtypeStruct((2, _NPAD, _D), jnp.float32),
        mesh=mesh,
        scratch_types=(
            pltpu.VMEM_SHARED((_NPAD, _D), jnp.float32),
            pltpu.VMEM((_EW,), jnp.int32),
            pltpu.VMEM((_B, _D), jnp.float32),
            pltpu.VMEM((_B, _D), jnp.float32),
            pltpu.VMEM((_B,), jnp.int32),
            pltpu.VMEM((_B,), jnp.int32),
            pltpu.VMEM((32, _D), jnp.float32),
            pltpu.SemaphoreType.DMA,
            pltpu.SemaphoreType.DMA,
            pltpu.SemaphoreType.DMA,
            pltpu.SemaphoreType.DMA,
            pltpu.SemaphoreType.DMA,
        ),
    )
    deg = pl.kernel(
        _degrees_body,
        out_type=jax.ShapeDtypeStruct((2 * _NPAD,), jnp.float32),
        mesh=mesh,
        scratch_types=(
            pltpu.VMEM_SHARED((_NPAD,), jnp.float32),
            pltpu.VMEM((_EW,), jnp.int32),
            pltpu.VMEM((_B,), jnp.int32),
            pltpu.VMEM((_B,), jnp.int32),
            pltpu.VMEM((_B,), jnp.float32),
            pltpu.VMEM((_RPT,), jnp.float32),
            pltpu.SemaphoreType.DMA,
            pltpu.SemaphoreType.DMA,
        ),
    )
    return agg, deg


_BLK = 2000
_BLKG = 1000


def _gru_body(p0, p1, c0, c1, h, wih, whh, bih, bhh, out_h, out_sq):
    i = pl.program_id(0)
    cnt = c0[...] + c1[...]
    agg = (p0[0] + p1[0]) / (cnt + jnp.float32(_EPS))
    hh = h[...]
    dn = (((1,), (1,)), ((), ()))
    gi = lax.dot_general(agg, wih[...], dn, preferred_element_type=jnp.float32)
    gi = gi + bih[...]
    gh = lax.dot_general(hh, whh[...], dn, preferred_element_type=jnp.float32)
    gh = gh + bhh[...]
    r = jax.nn.sigmoid(gi[:, :_D] + gh[:, :_D])
    z = jax.nn.sigmoid(gi[:, _D:2 * _D] + gh[:, _D:2 * _D])
    n = jnp.tanh(gi[:, 2 * _D:] + r * gh[:, 2 * _D:])
    new = (jnp.float32(1.0) - z) * n + z * hh
    out_h[...] = new

    @pl.when(i == 0)
    def _():
        out_sq[...] = jnp.zeros((1, 1), jnp.float32)
    out_sq[...] += jnp.sum((new - hh) ** 2).reshape(1, 1)


def _gru_step(p0, p1, c0, c1, h, wih, whh, bih, bhh):
    row = lambda i: (i, i * 0)
    p0m = lambda i: (i * 0, i, i * 0)
    p1m = lambda i: (i * 0 + 1, i, i * 0)
    fixed = lambda i: (i * 0, i * 0)
    return pl.pallas_call(
        _gru_body,
        grid=(_N // _BLKG,),
        in_specs=[
            pl.BlockSpec((1, _BLKG, _D), p0m),
            pl.BlockSpec((1, _BLKG, _D), p1m),
            pl.BlockSpec((_BLKG, 1), row),
            pl.BlockSpec((_BLKG, 1), row),
            pl.BlockSpec((_BLKG, _D), row),
            pl.BlockSpec((3 * _D, _D), fixed),
            pl.BlockSpec((3 * _D, _D), fixed),
            pl.BlockSpec((1, 3 * _D), fixed),
            pl.BlockSpec((1, 3 * _D), fixed),
        ],
        out_specs=[
            pl.BlockSpec((_BLKG, _D), row),
            pl.BlockSpec((1, 1), fixed),
        ],
        out_shape=[
            jax.ShapeDtypeStruct((_N, _D), jnp.float32),
            jax.ShapeDtypeStruct((1, 1), jnp.float32),
        ],
    )(p0, p1, c0, c1, h, wih, whh, bih, bhh)


def _head_body(h, wo, bo, tg, mk, y_h, ps, ms):
    i = pl.program_id(0)
    dn = (((1,), (1,)), ((), ()))
    y = lax.dot_general(h[...], wo[...], dn, preferred_element_type=jnp.float32)
    y = y + bo[...]
    mkf = mk[...]
    ym = y * mkf
    col = lax.broadcasted_iota(jnp.int32, (_BLK, _D), 1)
    valid = col < _OUT
    yv = jnp.where(valid, ym, jnp.float32(-1e30))
    m = jnp.max(yv, axis=1, keepdims=True)
    e = jnp.where(valid, jnp.exp(ym - m), jnp.float32(0.0))
    se = jnp.sum(e, axis=1, keepdims=True)
    logp = ym - m - jnp.log(se)
    picked = jnp.sum(jnp.where(col == tg[...], logp, jnp.float32(0.0)), axis=1, keepdims=True)
    y_h[...] = jnp.where(valid, ym, jnp.float32(0.0))[:, :8]

    @pl.when(i == 0)
    def _():
        ps[...] = jnp.zeros((1, 1), jnp.float32)
        ms[...] = jnp.zeros((1, 1), jnp.float32)
    ps[...] += jnp.sum(picked * mkf).reshape(1, 1)
    ms[...] += jnp.sum(mkf).reshape(1, 1)


def _head(h, wo, bo, tg, mk):
    row = lambda i: (i, i * 0)
    fixed = lambda i: (i * 0, i * 0)
    return pl.pallas_call(
        _head_body,
        grid=(_N // _BLK,),
        in_specs=[
            pl.BlockSpec((_BLK, _D), row),
            pl.BlockSpec((_D, _D), fixed),
            pl.BlockSpec((1, _D), fixed),
            pl.BlockSpec((_BLK, 1), row),
            pl.BlockSpec((_BLK, 1), row),
        ],
        out_specs=[
            pl.BlockSpec((_BLK, 8), row),
            pl.BlockSpec((1, 1), fixed),
            pl.BlockSpec((1, 1), fixed),
        ],
        out_shape=[
            jax.ShapeDtypeStruct((_N, 8), jnp.float32),
            jax.ShapeDtypeStruct((1, 1), jnp.float32),
            jax.ShapeDtypeStruct((1, 1), jnp.float32),
        ],
    )(h, wo, bo, tg, mk)


def kernel(edge, feat, target, mask, W_ih, W_hh, b_ih, b_hh, W_out, b_out):
    e = edge[0].astype(jnp.int32)
    npe = _EPAD - _E
    ar = jnp.arange(npe, dtype=jnp.int32)
    src = jnp.concatenate([e[:, 0], ar % _N])
    dst1 = jnp.concatenate([e[:, 1], _N + (ar % (_NPAD - _N))])
    state = feat[0].astype(jnp.float32)
    W_ih = W_ih.astype(jnp.float32)
    W_hh = W_hh.astype(jnp.float32)
    W_out = W_out.astype(jnp.float32)
    b_out = b_out.astype(jnp.float32)
    tgt64 = target[0]
    mk = mask[0]
    mkf = mk.astype(jnp.float32).reshape(_N, 1)
    tgm = jnp.where(mk, tgt64, 0).astype(jnp.int32).reshape(_N, 1)

    _agg_step, _degrees = _sc_kernels()
    cc = _degrees(dst1)
    c0 = cc[:_N].reshape(_N, 1)
    c1 = cc[_NPAD:_NPAD + _N].reshape(_N, 1)

    bih = b_ih.astype(jnp.float32).reshape(1, 3 * _D)
    bhh = b_hh.astype(jnp.float32).reshape(1, 3 * _D)
    wo = jnp.pad(W_out, ((0, _D - _OUT), (0, 0)))
    bo = jnp.pad(b_out, (0, _D - _OUT)).reshape(1, _D)

    diffs = []
    for _ in range(_PROP):
        pp = _agg_step(state, src, dst1)
        new_state, sq = _gru_step(pp, pp, c0, c1, state,
                                  W_ih, W_hh, bih, bhh)
        diffs.append(jnp.sqrt(sq[0, 0]) / float(_N * _D))
        state = new_state

    ym_full, ps, ms = _head(state, wo, bo, tgm, mkf)
    y_m = ym_full[:, :_OUT].astype(jnp.float64)
    t_m = jnp.where(mk, tgt64, jnp.zeros_like(tgt64))
    loss = -(ps[0, 0] / ms[0, 0]).astype(jnp.float64)
    return y_m, t_m, jnp.stack(diffs).astype(jnp.float64), loss


# speedup vs baseline: 1.0806x; 1.0394x over previous
import functools

import jax
import jax.numpy as jnp
import numpy as np
from jax import lax
from jax.experimental import pallas as pl
from jax.experimental.pallas import tpu as pltpu
from jax.experimental.pallas import tpu_sc as plsc

_N, _E, _D, _OUT, _PROP = 10000, 320000, 128, 7, 5
_EPS = float(np.finfo(np.float32).eps)

_NC, _NS = 2, 16
_NW = _NC * _NS
_B = 128
_NBB = 80
_EW = _NBB * _B
_EPAD = _EW * _NW
_NPAD = 10240
_RPT = _NPAD // _NS


def _zero_fill_vmem(ref, rows, cols):
    zero16 = jnp.zeros((16,), jnp.float32)
    for r in range(rows):
        for c in range(cols // 16):
            ref[r, pl.ds(c * 16, 16)] = zero16


def _agg_step_body(state_h, src_h, dst_h, pp_h,
                   acc_sh, sidx_v, b0, b1, i0, i1, zrow_v,
                   g0, g1, s0, s1, zsem):
    cid = lax.axis_index("c")
    sid = lax.axis_index("s")
    wid = sid * jnp.int32(_NC) + cid
    base = sid * jnp.int32(_RPT)

    bufs = (b0, b1)
    ibufs = (i0, i1)
    gsems = (g0, g1)
    ssems = (s0, s1)

    pltpu.sync_copy(src_h.at[pl.ds(wid * jnp.int32(_EW), _EW)], sidx_v)

    _zero_fill_vmem(zrow_v, 32, _D)

    def zbody(k, _):
        pltpu.async_copy(zrow_v, acc_sh.at[pl.ds(base + k * jnp.int32(32), 32)],
                         zsem)
        return jnp.int32(0)
    lax.fori_loop(jnp.int32(0), jnp.int32(_RPT // 32), zbody, jnp.int32(0))

    def zdrain(k, _):
        pltpu.make_async_copy(
            zrow_v, acc_sh.at[pl.ds(base, 32)], zsem).wait()
        return jnp.int32(0)
    lax.fori_loop(jnp.int32(0), jnp.int32(_RPT // 32), zdrain, jnp.int32(0))
    plsc.subcore_barrier()

    def start_gather(j, u):
        pltpu.async_copy(
            dst_h.at[pl.ds(wid * jnp.int32(_EW) + j * jnp.int32(_B), _B)],
            ibufs[u], gsems[u])
        pltpu.async_copy(
            state_h.at[sidx_v.at[pl.ds(j * jnp.int32(_B), _B)]],
            bufs[u], gsems[u])

    def wait_gather(u):
        pltpu.make_async_copy(
            dst_h.at[pl.ds(jnp.int32(0), _B)], ibufs[u], gsems[u]).wait()
        pltpu.make_async_copy(
            state_h.at[sidx_v.at[pl.ds(jnp.int32(0), _B)]],
            bufs[u], gsems[u]).wait()

    def start_scatter(j, u):
        pltpu.async_copy(bufs[u], acc_sh.at[ibufs[u]], ssems[u], add=True)

    def wait_scatter(u):
        pltpu.make_async_copy(bufs[u], acc_sh.at[ibufs[u]], ssems[u]).wait()

    def body(k, _):
        for u in range(2):
            j = k * jnp.int32(2) + jnp.int32(u)

            @pl.when(k >= jnp.int32(1))
            def _():
                wait_scatter(u)
            start_gather(j, u)
            jb = j - jnp.int32(1)
            ub = 1 - u

            @pl.when(jb >= jnp.int32(0))
            def _():
                wait_gather(ub)
                start_scatter(jb, ub)
        return jnp.int32(0)
    lax.fori_loop(jnp.int32(0), jnp.int32(_NBB // 2), body, jnp.int32(0))

    wait_gather(1)
    start_scatter(jnp.int32(_NBB - 1), 1)
    for u in range(2):
        wait_scatter(u)
    plsc.subcore_barrier()

    pltpu.sync_copy(acc_sh.at[pl.ds(base, _RPT)],
                    pp_h.at[cid, pl.ds(base, _RPT)])


def _degrees_body(dst_h, cc_h, acc_sh, didx_v, j0, j1, ones_v, zs_v,
                  s0, s1):
    cid = lax.axis_index("c")
    sid = lax.axis_index("s")
    wid = sid * jnp.int32(_NC) + cid
    base = sid * jnp.int32(_RPT)

    ibufs = (j0, j1)
    ssems = (s0, s1)

    zero16 = jnp.zeros((16,), jnp.float32)
    one16 = jnp.ones((16,), jnp.float32)
    for k in range(-(-_RPT // 16)):
        zs_v[pl.ds(min(k * 16, _RPT - 16), 16)] = zero16
    for k in range(_B // 16):
        ones_v[pl.ds(k * 16, 16)] = one16

    pltpu.sync_copy(zs_v, acc_sh.at[pl.ds(base, _RPT)])

    def ldbody(k, _):
        off = k * jnp.int32(1024)
        pltpu.sync_copy(dst_h.at[pl.ds(wid * jnp.int32(_EW) + off, 1024)],
                        didx_v.at[pl.ds(off, 1024)])
        return jnp.int32(0)
    lax.fori_loop(jnp.int32(0), jnp.int32(_EW // 1024), ldbody, jnp.int32(0))
    plsc.subcore_barrier()

    def start_scatter(j, u):
        for t in range(_B // 16):
            ibufs[u][pl.ds(t * 16, 16)] = didx_v[
                pl.ds(j * jnp.int32(_B) + jnp.int32(t * 16), 16)]
        pltpu.async_copy(ones_v, acc_sh.at[ibufs[u]], ssems[u], add=True)

    def wait_scatter(u):
        pltpu.make_async_copy(ones_v, acc_sh.at[ibufs[u]], ssems[u]).wait()

    def body(k, _):
        for u in range(2):
            j = k * jnp.int32(2) + jnp.int32(u)

            @pl.when(k >= jnp.int32(1))
            def _():
                wait_scatter(u)
            start_scatter(j, u)
        return jnp.int32(0)
    lax.fori_loop(jnp.int32(0), jnp.int32(_NBB // 2), body, jnp.int32(0))
    for u in range(2):
        wait_scatter(u)
    plsc.subcore_barrier()

    obase = cid * jnp.int32(_NPAD) + base
    pltpu.sync_copy(acc_sh.at[pl.ds(base, _RPT)], zs_v)
    pltpu.sync_copy(zs_v, cc_h.at[pl.ds(obase, _RPT)])


@functools.lru_cache(maxsize=1)
def _sc_kernels():
    mesh = plsc.VectorSubcoreMesh(core_axis_name="c", subcore_axis_name="s",
                                  num_cores=_NC, num_subcores=_NS)
    agg = pl.kernel(
        _agg_step_body,
        out_type=jax.ShapeDtypeStruct((2, _NPAD, _D), jnp.float32),
        mesh=mesh,
        scratch_types=(
            pltpu.VMEM_SHARED((_NPAD, _D), jnp.float32),
            pltpu.VMEM((_EW,), jnp.int32),
            pltpu.VMEM((_B, _D), jnp.float32),
            pltpu.VMEM((_B, _D), jnp.float32),
            pltpu.VMEM((_B,), jnp.int32),
            pltpu.VMEM((_B,), jnp.int32),
            pltpu.VMEM((32, _D), jnp.float32),
            pltpu.SemaphoreType.DMA,
            pltpu.SemaphoreType.DMA,
            pltpu.SemaphoreType.DMA,
            pltpu.SemaphoreType.DMA,
            pltpu.SemaphoreType.DMA,
        ),
    )
    deg = pl.kernel(
        _degrees_body,
        out_type=jax.ShapeDtypeStruct((2 * _NPAD,), jnp.float32),
        mesh=mesh,
        scratch_types=(
            pltpu.VMEM_SHARED((_NPAD,), jnp.float32),
            pltpu.VMEM((_EW,), jnp.int32),
            pltpu.VMEM((_B,), jnp.int32),
            pltpu.VMEM((_B,), jnp.int32),
            pltpu.VMEM((_B,), jnp.float32),
            pltpu.VMEM((_RPT,), jnp.float32),
            pltpu.SemaphoreType.DMA,
            pltpu.SemaphoreType.DMA,
        ),
    )
    return agg, deg


_BLK = 2000
_BLKG = 2000


def _gru_body(p0, p1, c0, c1, h, wih, whh, bih, bhh, out_h, out_sq):
    i = pl.program_id(0)
    cnt = c0[...] + c1[...]
    agg = (p0[0] + p1[0]) / (cnt + jnp.float32(_EPS))
    hh = h[...]
    dn = (((1,), (1,)), ((), ()))
    gi = lax.dot_general(agg, wih[...], dn, preferred_element_type=jnp.float32)
    gi = gi + bih[...]
    gh = lax.dot_general(hh, whh[...], dn, preferred_element_type=jnp.float32)
    gh = gh + bhh[...]
    r = jax.nn.sigmoid(gi[:, :_D] + gh[:, :_D])
    z = jax.nn.sigmoid(gi[:, _D:2 * _D] + gh[:, _D:2 * _D])
    n = jnp.tanh(gi[:, 2 * _D:] + r * gh[:, 2 * _D:])
    new = (jnp.float32(1.0) - z) * n + z * hh
    out_h[...] = new

    @pl.when(i == 0)
    def _():
        out_sq[...] = jnp.zeros((1, 1), jnp.float32)
    out_sq[...] += jnp.sum((new - hh) ** 2).reshape(1, 1)


def _gru_step(p0, p1, c0, c1, h, wih, whh, bih, bhh):
    row = lambda i: (i, i * 0)
    p0m = lambda i: (i * 0, i, i * 0)
    p1m = lambda i: (i * 0 + 1, i, i * 0)
    fixed = lambda i: (i * 0, i * 0)
    return pl.pallas_call(
        _gru_body,
        grid=(_N // _BLKG,),
        in_specs=[
            pl.BlockSpec((1, _BLKG, _D), p0m),
            pl.BlockSpec((1, _BLKG, _D), p1m),
            pl.BlockSpec((_BLKG, 1), row),
            pl.BlockSpec((_BLKG, 1), row),
            pl.BlockSpec((_BLKG, _D), row),
            pl.BlockSpec((3 * _D, _D), fixed),
            pl.BlockSpec((3 * _D, _D), fixed),
            pl.BlockSpec((1, 3 * _D), fixed),
            pl.BlockSpec((1, 3 * _D), fixed),
        ],
        out_specs=[
            pl.BlockSpec((_BLKG, _D), row),
            pl.BlockSpec((1, 1), fixed),
        ],
        out_shape=[
            jax.ShapeDtypeStruct((_N, _D), jnp.float32),
            jax.ShapeDtypeStruct((1, 1), jnp.float32),
        ],
    )(p0, p1, c0, c1, h, wih, whh, bih, bhh)


def _head_body(h, wo, bo, tg, mk, y_h, ps, ms):
    i = pl.program_id(0)
    dn = (((1,), (1,)), ((), ()))
    y = lax.dot_general(h[...], wo[...], dn, preferred_element_type=jnp.float32)
    y = y + bo[...]
    mkf = mk[...]
    ym = y * mkf
    col = lax.broadcasted_iota(jnp.int32, (_BLK, _D), 1)
    valid = col < _OUT
    yv = jnp.where(valid, ym, jnp.float32(-1e30))
    m = jnp.max(yv, axis=1, keepdims=True)
    e = jnp.where(valid, jnp.exp(ym - m), jnp.float32(0.0))
    se = jnp.sum(e, axis=1, keepdims=True)
    logp = ym - m - jnp.log(se)
    picked = jnp.sum(jnp.where(col == tg[...], logp, jnp.float32(0.0)), axis=1, keepdims=True)
    y_h[...] = jnp.where(valid, ym, jnp.float32(0.0))[:, :8]

    @pl.when(i == 0)
    def _():
        ps[...] = jnp.zeros((1, 1), jnp.float32)
        ms[...] = jnp.zeros((1, 1), jnp.float32)
    ps[...] += jnp.sum(picked * mkf).reshape(1, 1)
    ms[...] += jnp.sum(mkf).reshape(1, 1)


def _head(h, wo, bo, tg, mk):
    row = lambda i: (i, i * 0)
    fixed = lambda i: (i * 0, i * 0)
    return pl.pallas_call(
        _head_body,
        grid=(_N // _BLK,),
        in_specs=[
            pl.BlockSpec((_BLK, _D), row),
            pl.BlockSpec((_D, _D), fixed),
            pl.BlockSpec((1, _D), fixed),
            pl.BlockSpec((_BLK, 1), row),
            pl.BlockSpec((_BLK, 1), row),
        ],
        out_specs=[
            pl.BlockSpec((_BLK, 8), row),
            pl.BlockSpec((1, 1), fixed),
            pl.BlockSpec((1, 1), fixed),
        ],
        out_shape=[
            jax.ShapeDtypeStruct((_N, 8), jnp.float32),
            jax.ShapeDtypeStruct((1, 1), jnp.float32),
            jax.ShapeDtypeStruct((1, 1), jnp.float32),
        ],
    )(h, wo, bo, tg, mk)


def kernel(edge, feat, target, mask, W_ih, W_hh, b_ih, b_hh, W_out, b_out):
    e = edge[0].astype(jnp.int32)
    npe = _EPAD - _E
    ar = jnp.arange(npe, dtype=jnp.int32)
    src = jnp.concatenate([e[:, 0], ar % _N])
    dst1 = jnp.concatenate([e[:, 1], _N + (ar % (_NPAD - _N))])
    state = feat[0].astype(jnp.float32)
    W_ih = W_ih.astype(jnp.float32)
    W_hh = W_hh.astype(jnp.float32)
    W_out = W_out.astype(jnp.float32)
    b_out = b_out.astype(jnp.float32)
    tgt64 = target[0]
    mk = mask[0]
    mkf = mk.astype(jnp.float32).reshape(_N, 1)
    tgm = jnp.where(mk, tgt64, 0).astype(jnp.int32).reshape(_N, 1)

    _agg_step, _degrees = _sc_kernels()
    cc = _degrees(dst1)
    c0 = cc[:_N].reshape(_N, 1)
    c1 = cc[_NPAD:_NPAD + _N].reshape(_N, 1)

    bih = b_ih.astype(jnp.float32).reshape(1, 3 * _D)
    bhh = b_hh.astype(jnp.float32).reshape(1, 3 * _D)
    wo = jnp.pad(W_out, ((0, _D - _OUT), (0, 0)))
    bo = jnp.pad(b_out, (0, _D - _OUT)).reshape(1, _D)

    diffs = []
    for _ in range(_PROP):
        pp = _agg_step(state, src, dst1)
        new_state, sq = _gru_step(pp, pp, c0, c1, state,
                                  W_ih, W_hh, bih, bhh)
        diffs.append(jnp.sqrt(sq[0, 0]) / float(_N * _D))
        state = new_state

    ym_full, ps, ms = _head(state, wo, bo, tgm, mkf)
    y_m = ym_full[:, :_OUT].astype(jnp.float64)
    t_m = jnp.where(mk, tgt64, jnp.zeros_like(tgt64))
    loss = -(ps[0, 0] / ms[0, 0]).astype(jnp.float64)
    return y_m, t_m, jnp.stack(diffs).astype(jnp.float64), loss


# speedup vs baseline: 1.0901x; 1.0088x over previous
import functools

import jax
import jax.numpy as jnp
import numpy as np
from jax import lax
from jax.experimental import pallas as pl
from jax.experimental.pallas import tpu as pltpu
from jax.experimental.pallas import tpu_sc as plsc

_N, _E, _D, _OUT, _PROP = 10000, 320000, 128, 7, 5
_EPS = float(np.finfo(np.float32).eps)

_NC, _NS = 2, 16
_NW = _NC * _NS
_B = 128
_NBB = 80
_EW = _NBB * _B
_EPAD = _EW * _NW
_NPAD = 10240
_RPT = _NPAD // _NS


def _zero_fill_vmem(ref, rows, cols):
    zero16 = jnp.zeros((16,), jnp.float32)
    for r in range(rows):
        for c in range(cols // 16):
            ref[r, pl.ds(c * 16, 16)] = zero16


def _agg_step_body(state_h, src_h, dst_h, pp_h,
                   acc_sh, sidx_v, b0, b1, i0, i1, zrow_v,
                   g0, g1, s0, s1, zsem):
    cid = lax.axis_index("c")
    sid = lax.axis_index("s")
    wid = sid * jnp.int32(_NC) + cid
    base = sid * jnp.int32(_RPT)

    bufs = (b0, b1)
    ibufs = (i0, i1)
    gsems = (g0, g1)
    ssems = (s0, s1)

    _zero_fill_vmem(zrow_v, 32, _D)

    def zbody(k, _):
        pltpu.async_copy(zrow_v, acc_sh.at[pl.ds(base + k * jnp.int32(32), 32)],
                         zsem)
        return jnp.int32(0)
    lax.fori_loop(jnp.int32(0), jnp.int32(_RPT // 32), zbody, jnp.int32(0))

    pltpu.sync_copy(src_h.at[pl.ds(wid * jnp.int32(_EW), _EW)], sidx_v)

    def zdrain(k, _):
        pltpu.make_async_copy(
            zrow_v, acc_sh.at[pl.ds(base, 32)], zsem).wait()
        return jnp.int32(0)
    lax.fori_loop(jnp.int32(0), jnp.int32(_RPT // 32), zdrain, jnp.int32(0))
    plsc.subcore_barrier()

    def start_gather(j, u):
        pltpu.async_copy(
            dst_h.at[pl.ds(wid * jnp.int32(_EW) + j * jnp.int32(_B), _B)],
            ibufs[u], gsems[u])
        pltpu.async_copy(
            state_h.at[sidx_v.at[pl.ds(j * jnp.int32(_B), _B)]],
            bufs[u], gsems[u])

    def wait_gather(u):
        pltpu.make_async_copy(
            dst_h.at[pl.ds(jnp.int32(0), _B)], ibufs[u], gsems[u]).wait()
        pltpu.make_async_copy(
            state_h.at[sidx_v.at[pl.ds(jnp.int32(0), _B)]],
            bufs[u], gsems[u]).wait()

    def start_scatter(j, u):
        pltpu.async_copy(bufs[u], acc_sh.at[ibufs[u]], ssems[u], add=True)

    def wait_scatter(u):
        pltpu.make_async_copy(bufs[u], acc_sh.at[ibufs[u]], ssems[u]).wait()

    def body(k, _):
        for u in range(2):
            j = k * jnp.int32(2) + jnp.int32(u)

            @pl.when(k >= jnp.int32(1))
            def _():
                wait_scatter(u)
            start_gather(j, u)
            jb = j - jnp.int32(1)
            ub = 1 - u

            @pl.when(jb >= jnp.int32(0))
            def _():
                wait_gather(ub)
                start_scatter(jb, ub)
        return jnp.int32(0)
    lax.fori_loop(jnp.int32(0), jnp.int32(_NBB // 2), body, jnp.int32(0))

    wait_gather(1)
    start_scatter(jnp.int32(_NBB - 1), 1)
    for u in range(2):
        wait_scatter(u)
    plsc.subcore_barrier()

    pltpu.sync_copy(acc_sh.at[pl.ds(base, _RPT)],
                    pp_h.at[cid, pl.ds(base, _RPT)])


def _degrees_body(dst_h, cc_h, acc_sh, didx_v, j0, j1, ones_v, zs_v,
                  s0, s1):
    cid = lax.axis_index("c")
    sid = lax.axis_index("s")
    wid = sid * jnp.int32(_NC) + cid
    base = sid * jnp.int32(_RPT)

    ibufs = (j0, j1)
    ssems = (s0, s1)

    zero16 = jnp.zeros((16,), jnp.float32)
    one16 = jnp.ones((16,), jnp.float32)
    for k in range(-(-_RPT // 16)):
        zs_v[pl.ds(min(k * 16, _RPT - 16), 16)] = zero16
    for k in range(_B // 16):
        ones_v[pl.ds(k * 16, 16)] = one16

    pltpu.sync_copy(zs_v, acc_sh.at[pl.ds(base, _RPT)])

    def ldbody(k, _):
        off = k * jnp.int32(1024)
        pltpu.sync_copy(dst_h.at[pl.ds(wid * jnp.int32(_EW) + off, 1024)],
                        didx_v.at[pl.ds(off, 1024)])
        return jnp.int32(0)
    lax.fori_loop(jnp.int32(0), jnp.int32(_EW // 1024), ldbody, jnp.int32(0))
    plsc.subcore_barrier()

    def start_scatter(j, u):
        for t in range(_B // 16):
            ibufs[u][pl.ds(t * 16, 16)] = didx_v[
                pl.ds(j * jnp.int32(_B) + jnp.int32(t * 16), 16)]
        pltpu.async_copy(ones_v, acc_sh.at[ibufs[u]], ssems[u], add=True)

    def wait_scatter(u):
        pltpu.make_async_copy(ones_v, acc_sh.at[ibufs[u]], ssems[u]).wait()

    def body(k, _):
        for u in range(2):
            j = k * jnp.int32(2) + jnp.int32(u)

            @pl.when(k >= jnp.int32(1))
            def _():
                wait_scatter(u)
            start_scatter(j, u)
        return jnp.int32(0)
    lax.fori_loop(jnp.int32(0), jnp.int32(_NBB // 2), body, jnp.int32(0))
    for u in range(2):
        wait_scatter(u)
    plsc.subcore_barrier()

    obase = cid * jnp.int32(_NPAD) + base
    pltpu.sync_copy(acc_sh.at[pl.ds(base, _RPT)], zs_v)
    pltpu.sync_copy(zs_v, cc_h.at[pl.ds(obase, _RPT)])


@functools.lru_cache(maxsize=1)
def _sc_kernels():
    mesh = plsc.VectorSubcoreMesh(core_axis_name="c", subcore_axis_name="s",
                                  num_cores=_NC, num_subcores=_NS)
    agg = pl.kernel(
        _agg_step_body,
        out_type=jax.ShapeDtypeStruct((2, _NPAD, _D), jnp.float32),
        mesh=mesh,
        scratch_types=(
            pltpu.VMEM_SHARED((_NPAD, _D), jnp.float32),
            pltpu.VMEM((_EW,), jnp.int32),
            pltpu.VMEM((_B, _D), jnp.float32),
            pltpu.VMEM((_B, _D), jnp.float32),
            pltpu.VMEM((_B,), jnp.int32),
            pltpu.VMEM((_B,), jnp.int32),
            pltpu.VMEM((32, _D), jnp.float32),
            pltpu.SemaphoreType.DMA,
            pltpu.SemaphoreType.DMA,
            pltpu.SemaphoreType.DMA,
            pltpu.SemaphoreType.DMA,
            pltpu.SemaphoreType.DMA,
        ),
    )
    deg = pl.kernel(
        _degrees_body,
        out_type=jax.ShapeDtypeStruct((2 * _NPAD,), jnp.float32),
        mesh=mesh,
        scratch_types=(
            pltpu.VMEM_SHARED((_NPAD,), jnp.float32),
            pltpu.VMEM((_EW,), jnp.int32),
            pltpu.VMEM((_B,), jnp.int32),
            pltpu.VMEM((_B,), jnp.int32),
            pltpu.VMEM((_B,), jnp.float32),
            pltpu.VMEM((_RPT,), jnp.float32),
            pltpu.SemaphoreType.DMA,
            pltpu.SemaphoreType.DMA,
        ),
    )
    return agg, deg


_BLK = 2000
_BLKG = 2000


def _gru_body(p0, p1, c0, c1, h, wih, whh, bih, bhh, out_h, out_sq):
    i = pl.program_id(0)
    cnt = c0[...] + c1[...]
    agg = (p0[0] + p1[0]) / (cnt + jnp.float32(_EPS))
    hh = h[...]
    dn = (((1,), (1,)), ((), ()))
    gi = lax.dot_general(agg, wih[...], dn, preferred_element_type=jnp.float32)
    gi = gi + bih[...]
    gh = lax.dot_general(hh, whh[...], dn, preferred_element_type=jnp.float32)
    gh = gh + bhh[...]
    r = jax.nn.sigmoid(gi[:, :_D] + gh[:, :_D])
    z = jax.nn.sigmoid(gi[:, _D:2 * _D] + gh[:, _D:2 * _D])
    n = jnp.tanh(gi[:, 2 * _D:] + r * gh[:, 2 * _D:])
    new = (jnp.float32(1.0) - z) * n + z * hh
    out_h[...] = new

    @pl.when(i == 0)
    def _():
        out_sq[...] = jnp.zeros((1, 1), jnp.float32)
    out_sq[...] += jnp.sum((new - hh) ** 2).reshape(1, 1)


def _gru_step(p0, p1, c0, c1, h, wih, whh, bih, bhh):
    row = lambda i: (i, i * 0)
    p0m = lambda i: (i * 0, i, i * 0)
    p1m = lambda i: (i * 0 + 1, i, i * 0)
    fixed = lambda i: (i * 0, i * 0)
    return pl.pallas_call(
        _gru_body,
        grid=(_N // _BLKG,),
        in_specs=[
            pl.BlockSpec((1, _BLKG, _D), p0m),
            pl.BlockSpec((1, _BLKG, _D), p1m),
            pl.BlockSpec((_BLKG, 1), row),
            pl.BlockSpec((_BLKG, 1), row),
            pl.BlockSpec((_BLKG, _D), row),
            pl.BlockSpec((3 * _D, _D), fixed),
            pl.BlockSpec((3 * _D, _D), fixed),
            pl.BlockSpec((1, 3 * _D), fixed),
            pl.BlockSpec((1, 3 * _D), fixed),
        ],
        out_specs=[
            pl.BlockSpec((_BLKG, _D), row),
            pl.BlockSpec((1, 1), fixed),
        ],
        out_shape=[
            jax.ShapeDtypeStruct((_N, _D), jnp.float32),
            jax.ShapeDtypeStruct((1, 1), jnp.float32),
        ],
    )(p0, p1, c0, c1, h, wih, whh, bih, bhh)


def _head_body(h, wo, bo, tg, mk, y_h, ps, ms):
    i = pl.program_id(0)
    dn = (((1,), (1,)), ((), ()))
    y = lax.dot_general(h[...], wo[...], dn, preferred_element_type=jnp.float32)
    y = y + bo[...]
    mkf = mk[...]
    ym = y * mkf
    col = lax.broadcasted_iota(jnp.int32, (_BLK, _D), 1)
    valid = col < _OUT
    yv = jnp.where(valid, ym, jnp.float32(-1e30))
    m = jnp.max(yv, axis=1, keepdims=True)
    e = jnp.where(valid, jnp.exp(ym - m), jnp.float32(0.0))
    se = jnp.sum(e, axis=1, keepdims=True)
    logp = ym - m - jnp.log(se)
    picked = jnp.sum(jnp.where(col == tg[...], logp, jnp.float32(0.0)), axis=1, keepdims=True)
    y_h[...] = jnp.where(valid, ym, jnp.float32(0.0))[:, :8]

    @pl.when(i == 0)
    def _():
        ps[...] = jnp.zeros((1, 1), jnp.float32)
        ms[...] = jnp.zeros((1, 1), jnp.float32)
    ps[...] += jnp.sum(picked * mkf).reshape(1, 1)
    ms[...] += jnp.sum(mkf).reshape(1, 1)


def _head(h, wo, bo, tg, mk):
    row = lambda i: (i, i * 0)
    fixed = lambda i: (i * 0, i * 0)
    return pl.pallas_call(
        _head_body,
        grid=(_N // _BLK,),
        in_specs=[
            pl.BlockSpec((_BLK, _D), row),
            pl.BlockSpec((_D, _D), fixed),
            pl.BlockSpec((1, _D), fixed),
            pl.BlockSpec((_BLK, 1), row),
            pl.BlockSpec((_BLK, 1), row),
        ],
        out_specs=[
            pl.BlockSpec((_BLK, 8), row),
            pl.BlockSpec((1, 1), fixed),
            pl.BlockSpec((1, 1), fixed),
        ],
        out_shape=[
            jax.ShapeDtypeStruct((_N, 8), jnp.float32),
            jax.ShapeDtypeStruct((1, 1), jnp.float32),
            jax.ShapeDtypeStruct((1, 1), jnp.float32),
        ],
    )(h, wo, bo, tg, mk)


def kernel(edge, feat, target, mask, W_ih, W_hh, b_ih, b_hh, W_out, b_out):
    e = edge[0].astype(jnp.int32)
    npe = _EPAD - _E
    ar = jnp.arange(npe, dtype=jnp.int32)
    src = jnp.concatenate([e[:, 0], ar % _N])
    dst1 = jnp.concatenate([e[:, 1], _N + (ar % (_NPAD - _N))])
    state = feat[0].astype(jnp.float32)
    W_ih = W_ih.astype(jnp.float32)
    W_hh = W_hh.astype(jnp.float32)
    W_out = W_out.astype(jnp.float32)
    b_out = b_out.astype(jnp.float32)
    tgt64 = target[0]
    mk = mask[0]
    mkf = mk.astype(jnp.float32).reshape(_N, 1)
    tgm = jnp.where(mk, tgt64, 0).astype(jnp.int32).reshape(_N, 1)

    _agg_step, _degrees = _sc_kernels()
    cc = _degrees(dst1)
    c0 = cc[:_N].reshape(_N, 1)
    c1 = cc[_NPAD:_NPAD + _N].reshape(_N, 1)

    bih = b_ih.astype(jnp.float32).reshape(1, 3 * _D)
    bhh = b_hh.astype(jnp.float32).reshape(1, 3 * _D)
    wo = jnp.pad(W_out, ((0, _D - _OUT), (0, 0)))
    bo = jnp.pad(b_out, (0, _D - _OUT)).reshape(1, _D)

    diffs = []
    for _ in range(_PROP):
        pp = _agg_step(state, src, dst1)
        new_state, sq = _gru_step(pp, pp, c0, c1, state,
                                  W_ih, W_hh, bih, bhh)
        diffs.append(jnp.sqrt(sq[0, 0]) / float(_N * _D))
        state = new_state

    ym_full, ps, ms = _head(state, wo, bo, tgm, mkf)
    y_m = ym_full[:, :_OUT].astype(jnp.float64)
    t_m = jnp.where(mk, tgt64, jnp.zeros_like(tgt64))
    loss = -(ps[0, 0] / ms[0, 0]).astype(jnp.float64)
    return y_m, t_m, jnp.stack(diffs).astype(jnp.float64), loss
